# baseline (speedup 1.0000x reference)
import functools

import jax
import jax.numpy as jnp
from jax import lax
from jax.experimental import pallas as pl
from jax.experimental.pallas import tpu as pltpu
from jax.experimental.pallas import tpu_sc as plsc

_D = 64
_B = 4096
_H = 200

_NC = 2
_NS = 16
_NW = _NC * _NS
_BB = _B // _NW
_L = 16
_BLK = 8 * 8 * 128


def _embed_gather(weight, idx_t):
  mesh = plsc.VectorSubcoreMesh(core_axis_name="c", subcore_axis_name="s")

  @functools.partial(
      pl.kernel,
      mesh=mesh,
      out_type=jax.ShapeDtypeStruct((_H, 8, _NW, 1024), jnp.float32),
      compiler_params=pltpu.CompilerParams(
          use_tc_tiling_on_sc=False, needs_layout_passes=False),
      scratch_types=[
          pltpu.VMEM((_H, _BB), jnp.int32),
          pltpu.VMEM((_BB, _D), jnp.float32),
          pltpu.VMEM((_BB, _D), jnp.float32),
          pltpu.VMEM((8, 1024), jnp.float32),
          pltpu.VMEM((8, 1024), jnp.float32),
          pltpu.SemaphoreType.DMA,
          pltpu.SemaphoreType.DMA,
          pltpu.SemaphoreType.DMA,
          pltpu.SemaphoreType.DMA,
          pltpu.SemaphoreType.DMA,
      ],
  )
  def k(table_hbm, idx_hbm, out_hbm, idx_v, ga, gb, ta, tb,
        ig, sg0, sg1, so0, so1):
    wid = lax.axis_index("s") * _NC + lax.axis_index("c")
    pltpu.async_copy(
        idx_hbm.at[:, pl.ds(wid * _BB, _BB)], idx_v, ig).wait()

    rowidx = [lax.iota(jnp.int32, _L) + (g * _L) for g in range(8)]

    def fire_gather(h, buf, sem):
      pltpu.async_copy(table_hbm.at[idx_v.at[h]], buf, sem)

    def drain_gather(h, buf, sem):
      pltpu.make_async_copy(table_hbm.at[idx_v.at[h]], buf, sem).wait()

    def transpose(gbuf, tbuf):
      for big in range(8):
        for r in range(8):
          d = jnp.broadcast_to(jnp.int32(big * 8 + r), (_L,))
          for g in range(8):
            vals = plsc.load_gather(gbuf, [rowidx[g], d])
            tbuf[big, pl.ds(r * 128 + g * _L, _L)] = vals

    def fire_out(h, tbuf, sem):
      pltpu.async_copy(tbuf, out_hbm.at[h, :, wid], sem)

    def drain_out(h, tbuf, sem):
      pltpu.make_async_copy(tbuf, out_hbm.at[h, :, wid], sem).wait()

    fire_gather(0, ga, sg0)

    def body(t, carry):
      a = 2 * t
      b = a + 1

      @pl.when(t > 0)
      def _():
        drain_out(a - 2, ta, so0)

      fire_gather(b, gb, sg1)
      drain_gather(a, ga, sg0)
      transpose(ga, ta)
      fire_out(a, ta, so0)

      @pl.when(t < _H // 2 - 1)
      def _():
        fire_gather(a + 2, ga, sg0)

      @pl.when(t > 0)
      def _():
        drain_out(b - 2, tb, so1)

      drain_gather(b, gb, sg1)
      transpose(gb, tb)
      fire_out(b, tb, so1)
      return carry

    lax.fori_loop(0, _H // 2, body, 0)
    drain_out(_H - 2, ta, so0)
    drain_out(_H - 1, tb, so1)

  return k(weight, idx_t)


def kernel(indices, weight):
  idx_t = indices.T
  out5 = _embed_gather(weight, idx_t)
  out = (out5.reshape(_H, 8, _NW, 8, 128)
         .transpose(2, 4, 0, 1, 3)
         .reshape(_B, _H, _D))
  return out


# speedup vs baseline: 2.1658x; 2.1658x over previous
import functools

import jax
import jax.numpy as jnp
from jax import lax
from jax.experimental import pallas as pl
from jax.experimental.pallas import tpu as pltpu
from jax.experimental.pallas import tpu_sc as plsc

_D = 64
_B = 4096
_H = 200

_NC = 2
_NS = 16
_NW = _NC * _NS
_TOTAL = _B * _H
_PER_W = _TOTAL // _NW
_CHUNK = 128
_NCHUNK = _PER_W // _CHUNK
_K = 5
_GROUP = _K * _CHUNK
_NG = _NCHUNK // _K
_NIT = _NG // 2


def _embed_gather(weight, idx3):
  mesh = plsc.VectorSubcoreMesh(core_axis_name="c", subcore_axis_name="s")

  @functools.partial(
      pl.kernel,
      mesh=mesh,
      out_type=jax.ShapeDtypeStruct((_TOTAL, 2 * _D), jnp.float32),
      compiler_params=pltpu.CompilerParams(
          use_tc_tiling_on_sc=False, needs_layout_passes=False),
      scratch_types=[
          pltpu.VMEM((_NCHUNK, _CHUNK), jnp.int32),
          pltpu.VMEM((_GROUP, _D), jnp.float32),
          pltpu.VMEM((_GROUP, _D), jnp.float32),
          pltpu.SemaphoreType.DMA,
          pltpu.SemaphoreType.DMA,
          pltpu.SemaphoreType.DMA,
          pltpu.SemaphoreType.DMA,
      ],
  )
  def k(table_hbm, idx_hbm, out_hbm, idx_v, bufa, bufb, g0, g1, o0, o1):
    wid = lax.axis_index("s") * _NC + lax.axis_index("c")
    base = wid * _PER_W
    pltpu.sync_copy(idx_hbm.at[wid], idx_v)

    def fire_gather(group, buf, sem):
      for j in range(_K):
        pltpu.async_copy(
            table_hbm.at[idx_v.at[group * _K + j]],
            buf.at[pl.ds(j * _CHUNK, _CHUNK)], sem)

    def drain_gather(group, buf, sem):
      for j in range(_K):
        pltpu.make_async_copy(
            table_hbm.at[idx_v.at[group * _K + j]],
            buf.at[pl.ds(j * _CHUNK, _CHUNK)], sem).wait()

    def fire_out(group, buf, sem):
      pltpu.async_copy(
          buf,
          out_hbm.at[pl.ds(base + group * _GROUP, _GROUP), pl.ds(0, _D)],
          sem)

    def drain_out(group, buf, sem):
      pltpu.make_async_copy(
          buf,
          out_hbm.at[pl.ds(base + group * _GROUP, _GROUP), pl.ds(0, _D)],
          sem).wait()

    fire_gather(0, bufa, g0)

    def body(t, carry):
      a = 2 * t
      b = a + 1

      @pl.when(t > 0)
      def _():
        drain_out(b - 2, bufb, o1)

      fire_gather(b, bufb, g1)
      drain_gather(a, bufa, g0)
      fire_out(a, bufa, o0)
      drain_out(a, bufa, o0)

      @pl.when(t < _NIT - 1)
      def _():
        fire_gather(a + 2, bufa, g0)

      drain_gather(b, bufb, g1)
      fire_out(b, bufb, o1)
      return carry

    lax.fori_loop(0, _NIT, body, 0)
    drain_out(_NG - 1, bufb, o1)

  return k(weight, idx3)


def kernel(indices, weight):
  idx3 = indices.reshape(_NW, _NCHUNK, _CHUNK)
  out2 = _embed_gather(weight, idx3)
  return out2.reshape(_B, _H, 2 * _D)[:, :, :_D]


# speedup vs baseline: 2.1722x; 1.0029x over previous
import functools

import jax
import jax.numpy as jnp
from jax import lax
from jax.experimental import pallas as pl
from jax.experimental.pallas import tpu as pltpu
from jax.experimental.pallas import tpu_sc as plsc

_D = 64
_B = 4096
_H = 200

_NC = 2
_NS = 16
_NW = _NC * _NS
_TOTAL = _B * _H
_PER_W = _TOTAL // _NW
_CHUNK = 128
_NCHUNK = _PER_W // _CHUNK
_K = 5
_GROUP = _K * _CHUNK
_NG = _NCHUNK // _K
_NIT = _NG // 2


def _embed_gather(weight, idx3):
  mesh = plsc.VectorSubcoreMesh(core_axis_name="c", subcore_axis_name="s")

  @functools.partial(
      pl.kernel,
      mesh=mesh,
      out_type=jax.ShapeDtypeStruct((_TOTAL, 2 * _D), jnp.float32),
      compiler_params=pltpu.CompilerParams(
          use_tc_tiling_on_sc=False, needs_layout_passes=False),
      scratch_types=[
          pltpu.VMEM((_NCHUNK, _CHUNK), jnp.int32),
          pltpu.VMEM((_GROUP, _D), jnp.float32),
          pltpu.VMEM((_GROUP, _D), jnp.float32),
          pltpu.SemaphoreType.DMA,
          pltpu.SemaphoreType.DMA,
      ],
  )
  def k(table_hbm, idx_hbm, out_hbm, idx_v, bufa, bufb, g0, g1):
    wid = lax.axis_index("s") * _NC + lax.axis_index("c")
    base = wid * _PER_W
    pltpu.sync_copy(idx_hbm.at[wid], idx_v)

    def fire_gather(group, buf, sem):
      for j in range(_K):
        pltpu.async_copy(
            table_hbm.at[idx_v.at[group * _K + j]],
            buf.at[pl.ds(j * _CHUNK, _CHUNK)], sem)

    def drain_gather(group, buf, sem):
      for j in range(_K):
        pltpu.make_async_copy(
            table_hbm.at[idx_v.at[group * _K + j]],
            buf.at[pl.ds(j * _CHUNK, _CHUNK)], sem).wait()

    def sync_out(group, buf):
      pltpu.sync_copy(
          buf,
          out_hbm.at[pl.ds(base + group * _GROUP, _GROUP), pl.ds(0, _D)])

    fire_gather(0, bufa, g0)

    def body(t, carry):
      a = 2 * t
      b = a + 1

      fire_gather(b, bufb, g1)
      drain_gather(a, bufa, g0)
      sync_out(a, bufa)

      @pl.when(t < _NIT - 1)
      def _():
        fire_gather(a + 2, bufa, g0)

      drain_gather(b, bufb, g1)
      sync_out(b, bufb)
      return carry

    lax.fori_loop(0, _NIT, body, 0)

  return k(weight, idx3)


def kernel(indices, weight):
  idx3 = indices.reshape(_NW, _NCHUNK, _CHUNK)
  out2 = _embed_gather(weight, idx3)
  return out2.reshape(_B, _H, 2 * _D)[:, :, :_D]
